# Initial kernel scaffold; baseline (speedup 1.0000x reference)
import jax
import jax.numpy as jnp
from jax.experimental import pallas as pl


def kernel(z_author, z_hotel, edge_label_index, W1, b1, W2, b2):
    raise NotImplementedError("write your pallas kernel here")


# speedup vs baseline: 3.1744x; 3.1744x over previous
import functools

import jax
import jax.numpy as jnp
from jax import lax
from jax.experimental import pallas as pl
from jax.experimental.pallas import tpu as pltpu
from jax.experimental.pallas import tpu_sc as plsc

E = 320000
H = 128

NC = 2
NS = 16
NW = NC * NS
EPW = E // NW
C = 80
NCHUNK = EPW // C


def _gather_body(za_hbm, zh_hbm, row_hbm, col_hbm, out_hbm,
                 rowv, colv, bufa, bufh, sem_a, sem_h):
    wid = lax.axis_index("s") * NC + lax.axis_index("c")
    base = wid * EPW
    pltpu.sync_copy(row_hbm.at[pl.ds(base, EPW)], rowv)
    pltpu.sync_copy(col_hbm.at[pl.ds(base, EPW)], colv)

    def body(j, carry):
        off = pl.multiple_of(j * C, 8)
        cpa = pltpu.async_copy(za_hbm.at[rowv.at[pl.ds(off, C)]], bufa, sem_a)
        cph = pltpu.async_copy(zh_hbm.at[colv.at[pl.ds(off, C)]], bufh, sem_h)
        cpa.wait()
        cph.wait()
        sa = pltpu.async_copy(bufa, out_hbm.at[pl.ds(base + off, C), pl.ds(0, H)], sem_a)
        sh = pltpu.async_copy(bufh, out_hbm.at[pl.ds(base + off, C), pl.ds(H, H)], sem_h)
        sa.wait()
        sh.wait()
        return carry

    lax.fori_loop(0, NCHUNK, body, 0)


@functools.cache
def _gather_fn():
    return functools.partial(
        pl.kernel,
        mesh=plsc.VectorSubcoreMesh(core_axis_name="c", subcore_axis_name="s"),
        out_type=jax.ShapeDtypeStruct((E, 2 * H), jnp.float32),
        scratch_types=[
            pltpu.VMEM((EPW,), jnp.int32),
            pltpu.VMEM((EPW,), jnp.int32),
            pltpu.VMEM((C, H), jnp.float32),
            pltpu.VMEM((C, H), jnp.float32),
            pltpu.SemaphoreType.DMA,
            pltpu.SemaphoreType.DMA,
        ],
    )(_gather_body)


BLK = 2560


def _mlp_body(z1_ref, w1_ref, b1_ref, w2_ref, b2_ref, z2_ref, z3_ref):
    x = z1_ref[...]
    h = jnp.dot(x, w1_ref[...], preferred_element_type=jnp.float32)
    h = jnp.maximum(h + b1_ref[...][None, :], 0.0)
    z2_ref[...] = h
    z3_ref[...] = (jnp.sum(h * w2_ref[...], axis=1) + b2_ref[0])[:, None]


def _mlp(z1, W1, b1, w2t, b2):
    grid = (E // BLK,)
    return pl.pallas_call(
        _mlp_body,
        grid=grid,
        in_specs=[
            pl.BlockSpec((BLK, 2 * H), lambda i: (i, 0)),
            pl.BlockSpec((2 * H, H), lambda i: (0, 0)),
            pl.BlockSpec((H,), lambda i: (0,)),
            pl.BlockSpec((1, H), lambda i: (0, 0)),
            pl.BlockSpec((1,), lambda i: (0,)),
        ],
        out_specs=[
            pl.BlockSpec((BLK, H), lambda i: (i, 0)),
            pl.BlockSpec((BLK, 1), lambda i: (i, 0)),
        ],
        out_shape=[
            jax.ShapeDtypeStruct((E, H), jnp.float32),
            jax.ShapeDtypeStruct((E, 1), jnp.float32),
        ],
        compiler_params=pltpu.CompilerParams(
            dimension_semantics=("arbitrary",),
        ),
    )(z1, W1, b1, w2t, b2)


def kernel(z_author, z_hotel, edge_label_index, W1, b1, W2, b2):
    row = edge_label_index[0]
    col = edge_label_index[1]
    z1 = _gather_fn()(z_author, z_hotel, row, col)
    z2, z3 = _mlp(z1, W1, b1, W2.reshape(1, H), b2)
    return (z3.reshape(-1), (z1, z2))
